# baseline (speedup 1.0000x reference)
import functools

import jax
import jax.numpy as jnp
from jax import lax
from jax.experimental import pallas as pl
from jax.experimental.pallas import tpu as pltpu
from jax.experimental.pallas import tpu_sc as plsc

V = 30522
D = 768
G1, G2 = 2, 4
E = 8
K = 5
S = 2048
VT = 512
NV = (V + VT - 1) // VT


def _bf(a):
    return a.astype(jnp.bfloat16)


def _make_sc_gather():
    info = plsc.get_sparse_core_info()
    nc, ns = info.num_cores, info.num_subcores
    nw = nc * ns
    b_per_w = S // nw
    mesh = plsc.VectorSubcoreMesh(core_axis_name="c", subcore_axis_name="s")

    @functools.partial(
        pl.kernel,
        out_type=jax.ShapeDtypeStruct((S, D), jnp.float32),
        mesh=mesh,
        scratch_types=[
            pltpu.VMEM((b_per_w,), jnp.int32),
            pltpu.VMEM((b_per_w, D), jnp.float32),
            pltpu.SemaphoreType.DMA,
        ],
    )
    def gather_k(table_hbm, idx_hbm, out_hbm, idx_v, rows_v, sem):
        wid = lax.axis_index("s") * nc + lax.axis_index("c")
        base = wid * b_per_w
        pltpu.sync_copy(idx_hbm.at[pl.ds(base, b_per_w)], idx_v)
        pltpu.async_copy(table_hbm.at[idx_v], rows_v, sem).wait()
        pltpu.sync_copy(rows_v, out_hbm.at[pl.ds(base, b_per_w)])

    return gather_k


def _moe_body(x_ref, sc_ref, w1_ref, b1_ref, w2_ref, b2_ref, mask_ref,
              out_ref, gd_ref):
    e = pl.program_id(0)
    col = lax.broadcasted_iota(jnp.int32, (S, E), 1)

    @pl.when(e == 0)
    def _prologue():
        s = sc_ref[...]
        rank = jnp.zeros(s.shape, jnp.float32)
        for j in range(E):
            sj = s[:, j:j + 1]
            rank += jnp.where(sj > s, 1.0, 0.0)
            rank += jnp.where((sj == s) & (j < col), 1.0, 0.0)
        sm = jnp.where(rank < K, s, -jnp.inf)
        mx = jnp.max(sm, axis=1, keepdims=True)
        p = jnp.exp(sm - mx)
        gd_ref[...] = p / jnp.sum(p, axis=1, keepdims=True)

    xb = _bf(x_ref[...])
    h = jnp.dot(xb, _bf(w1_ref[0]), preferred_element_type=jnp.float32) \
        + b1_ref[0]
    h = jax.nn.gelu(h)
    y = jnp.dot(_bf(h), _bf(w2_ref[0]), preferred_element_type=jnp.float32) \
        + b2_ref[0]
    g = jnp.sum(jnp.where(col == e, gd_ref[...], 0.0), axis=1, keepdims=True)
    gy = _bf(g).astype(jnp.float32) * _bf(y).astype(jnp.float32)

    @pl.when(e == 0)
    def _init():
        out_ref[...] = gy

    @pl.when(e > 0)
    def _acc():
        out_ref[...] += gy

    @pl.when(e == E - 1)
    def _mask():
        out_ref[...] = out_ref[...] * mask_ref[...]


def _moe_pallas_args():
    full2d = pl.BlockSpec((S, D), lambda e: (0, 0))
    return dict(
        grid=(E,),
        in_specs=[
            full2d,
            pl.BlockSpec((S, E), lambda e: (0, 0)),
            pl.BlockSpec((1, D, D), lambda e: (e, 0, 0)),
            pl.BlockSpec((1, 1, D), lambda e: (e, 0, 0)),
            pl.BlockSpec((1, D, D), lambda e: (e, 0, 0)),
            pl.BlockSpec((1, 1, D), lambda e: (e, 0, 0)),
            pl.BlockSpec((S, 1), lambda e: (0, 0)),
        ],
        out_specs=full2d,
        out_shape=jax.ShapeDtypeStruct((S, D), jnp.float32),
        scratch_shapes=[pltpu.VMEM((S, E), jnp.float32)],
    )


def _head_body(x_ref, hw_ref, hb_ref, hg_ref, hbe_ref, dw_ref, db_ref,
               lab_ref, logits_ref, loss_ref, h_s, m_s, s_s, ll_s):
    j = pl.program_id(0)

    @pl.when(j == 0)
    def _prologue():
        hh = jnp.dot(_bf(x_ref[...]), _bf(hw_ref[...]),
                     preferred_element_type=jnp.float32) + hb_ref[...]
        hh = jax.nn.gelu(hh)
        mu = jnp.mean(hh, axis=1, keepdims=True)
        var = jnp.mean((hh - mu) ** 2, axis=1, keepdims=True)
        h_s[...] = ((hh - mu) / jnp.sqrt(var + 1e-5) * hg_ref[...]
                    + hbe_ref[...])
        m_s[...] = jnp.full((S, 1), -jnp.inf, jnp.float32)
        s_s[...] = jnp.zeros((S, 1), jnp.float32)
        ll_s[...] = jnp.zeros((S, 1), jnp.float32)

    logits = jnp.dot(_bf(h_s[...]), _bf(dw_ref[...]),
                     preferred_element_type=jnp.float32) + db_ref[...]
    logits_ref[0] = logits
    colg = j * VT + lax.broadcasted_iota(jnp.int32, (S, VT), 1)
    lg = jnp.where(colg < V, logits, -jnp.inf)
    m_old = m_s[...]
    m_new = jnp.maximum(m_old, jnp.max(lg, axis=1, keepdims=True))
    s_s[...] = (s_s[...] * jnp.exp(m_old - m_new)
                + jnp.sum(jnp.exp(lg - m_new), axis=1, keepdims=True))
    m_s[...] = m_new
    ll_s[...] += jnp.sum(jnp.where(colg == lab_ref[...], logits, 0.0),
                         axis=1, keepdims=True)

    @pl.when(j == NV - 1)
    def _fin():
        nll = m_s[...] + jnp.log(s_s[...]) - ll_s[...]
        loss_ref[...] = jnp.sum(nll, keepdims=True) / S


def _head_pallas_args():
    row = pl.BlockSpec((1, D), lambda j: (0, 0))
    return dict(
        grid=(NV,),
        in_specs=[
            pl.BlockSpec((S, D), lambda j: (0, 0)),
            pl.BlockSpec((D, D), lambda j: (0, 0)),
            row, row, row,
            pl.BlockSpec((D, VT), lambda j: (0, j)),
            pl.BlockSpec((1, VT), lambda j: (0, j)),
            pl.BlockSpec((S, 1), lambda j: (0, 0)),
        ],
        out_specs=[
            pl.BlockSpec((1, S, VT), lambda j: (0, 0, j)),
            pl.BlockSpec((1, 1), lambda j: (0, 0)),
        ],
        out_shape=[
            jax.ShapeDtypeStruct((1, S, V), jnp.float32),
            jax.ShapeDtypeStruct((1, 1), jnp.float32),
        ],
        scratch_shapes=[
            pltpu.VMEM((S, D), jnp.float32),
            pltpu.VMEM((S, 1), jnp.float32),
            pltpu.VMEM((S, 1), jnp.float32),
            pltpu.VMEM((S, 1), jnp.float32),
        ],
    )


def _forward(gather_fn, moe, head, input_ids, attention_mask, labels,
             word_emb, pos_emb, type_emb, emb_ln_g, emb_ln_b, Wg1, bg1, Wg2,
             bg2, W1, b1, W2, b2, head_w, head_b, head_ln_g, head_ln_b,
             dec_w, dec_b):
    ids = input_ids.reshape(S)
    gathered = gather_fn(word_emb, ids)
    pos_ids = jnp.clip(jnp.arange(S) + 2, 0, pos_emb.shape[0] - 1)
    e3 = (gathered.reshape(1, S, D) + pos_emb[pos_ids][None, :, :]
          + type_emb[0])
    m = jnp.mean(e3, axis=-1, keepdims=True)
    v = jnp.var(e3, axis=-1, keepdims=True)
    x = ((e3 - m) / jnp.sqrt(v + 1e-5) * emb_ln_g + emb_ln_b).reshape(S, D)
    mask = attention_mask.reshape(S, 1).astype(jnp.float32)

    for i in range(2):
        l1 = x @ Wg1[i] + bg1[i]
        l2 = x @ Wg2[i] + bg2[i]
        scores = (l1[:, :, None] + l2[:, None, :]).reshape(S, E)
        x = moe(x, scores, W1[i], b1[i].reshape(E, 1, D), W2[i],
                b2[i].reshape(E, 1, D), mask)

    logits, loss11 = head(x, head_w, head_b.reshape(1, D),
                          head_ln_g.reshape(1, D),
                          head_ln_b.reshape(1, D), dec_w,
                          dec_b.reshape(1, V), labels.reshape(S, 1))
    return (loss11[0, 0], logits, x.reshape(1, S, D))


def kernel(input_ids, attention_mask, labels, word_emb, pos_emb, type_emb,
           emb_ln_g, emb_ln_b, Wg1, bg1, Wg2, bg2, W1, b1, W2, b2, head_w,
           head_b, head_ln_g, head_ln_b, dec_w, dec_b):
    gather_fn = _make_sc_gather()
    moe = pl.pallas_call(_moe_body, **_moe_pallas_args())
    head = pl.pallas_call(_head_body, **_head_pallas_args())
    return _forward(gather_fn, moe, head, input_ids, attention_mask,
                    labels, word_emb, pos_emb, type_emb, emb_ln_g, emb_ln_b,
                    Wg1, bg1, Wg2, bg2, W1, b1, W2, b2, head_w, head_b,
                    head_ln_g, head_ln_b, dec_w, dec_b)


# speedup vs baseline: 1.0990x; 1.0990x over previous
import functools

import jax
import jax.numpy as jnp
from jax import lax
from jax.experimental import pallas as pl
from jax.experimental.pallas import tpu as pltpu
from jax.experimental.pallas import tpu_sc as plsc

V = 30522
D = 768
G1, G2 = 2, 4
E = 8
K = 5
S = 2048
VT = 1024
NV = (V + VT - 1) // VT


def _bf(a):
    return a.astype(jnp.bfloat16)


def _make_sc_gather():
    info = plsc.get_sparse_core_info()
    nc, ns = info.num_cores, info.num_subcores
    nw = nc * ns
    b_per_w = S // nw
    mesh = plsc.VectorSubcoreMesh(core_axis_name="c", subcore_axis_name="s")

    @functools.partial(
        pl.kernel,
        out_type=jax.ShapeDtypeStruct((S, D), jnp.float32),
        mesh=mesh,
        scratch_types=[
            pltpu.VMEM((b_per_w,), jnp.int32),
            pltpu.VMEM((b_per_w, D), jnp.float32),
            pltpu.SemaphoreType.DMA,
        ],
    )
    def gather_k(table_hbm, idx_hbm, out_hbm, idx_v, rows_v, sem):
        wid = lax.axis_index("s") * nc + lax.axis_index("c")
        base = wid * b_per_w
        pltpu.sync_copy(idx_hbm.at[pl.ds(base, b_per_w)], idx_v)
        pltpu.async_copy(table_hbm.at[idx_v], rows_v, sem).wait()
        pltpu.sync_copy(rows_v, out_hbm.at[pl.ds(base, b_per_w)])

    return gather_k


def _moe_body(xb_ref, sc_ref, w1_ref, b1_ref, w2_ref, b2_ref, mask_ref,
              out_ref, gd_ref):
    e = pl.program_id(0)
    col = lax.broadcasted_iota(jnp.int32, (S, E), 1)

    @pl.when(e == 0)
    def _prologue():
        s = sc_ref[...]
        rank = jnp.zeros(s.shape, jnp.float32)
        for j in range(E):
            sj = s[:, j:j + 1]
            rank += jnp.where(sj > s, 1.0, 0.0)
            rank += jnp.where((sj == s) & (j < col), 1.0, 0.0)
        sm = jnp.where(rank < K, s, -jnp.inf)
        mx = jnp.max(sm, axis=1, keepdims=True)
        p = jnp.exp(sm - mx)
        gd_ref[...] = p / jnp.sum(p, axis=1, keepdims=True)

    h = jnp.dot(xb_ref[...], _bf(w1_ref[0]),
                preferred_element_type=jnp.float32) + b1_ref[0]
    h = jax.nn.gelu(h)
    y = jnp.dot(_bf(h), _bf(w2_ref[0]), preferred_element_type=jnp.float32) \
        + b2_ref[0]
    g = jnp.sum(jnp.where(col == e, gd_ref[...], 0.0), axis=1, keepdims=True)
    gy = _bf(g).astype(jnp.float32) * _bf(y).astype(jnp.float32)

    @pl.when(e == 0)
    def _init():
        out_ref[...] = gy

    @pl.when(e > 0)
    def _acc():
        out_ref[...] += gy

    @pl.when(e == E - 1)
    def _mask():
        out_ref[...] = out_ref[...] * mask_ref[...]


def _moe_pallas_args():
    full2d = pl.BlockSpec((S, D), lambda e: (0, 0))
    return dict(
        grid=(E,),
        in_specs=[
            full2d,
            pl.BlockSpec((S, E), lambda e: (0, 0)),
            pl.BlockSpec((1, D, D), lambda e: (e, 0, 0)),
            pl.BlockSpec((1, 1, D), lambda e: (e, 0, 0)),
            pl.BlockSpec((1, D, D), lambda e: (e, 0, 0)),
            pl.BlockSpec((1, 1, D), lambda e: (e, 0, 0)),
            pl.BlockSpec((S, 1), lambda e: (0, 0)),
        ],
        out_specs=full2d,
        out_shape=jax.ShapeDtypeStruct((S, D), jnp.float32),
        scratch_shapes=[pltpu.VMEM((S, E), jnp.float32)],
    )


def _head_body(x_ref, hw_ref, hb_ref, hg_ref, hbe_ref, dw_ref, db_ref,
               lab_ref, logits_ref, loss_ref, h_s, m_s, s_s, ll_s):
    j = pl.program_id(0)

    @pl.when(j == 0)
    def _prologue():
        hh = jnp.dot(x_ref[...], hw_ref[...],
                     preferred_element_type=jnp.float32) + hb_ref[...]
        hh = jax.nn.gelu(hh)
        mu = jnp.mean(hh, axis=1, keepdims=True)
        var = jnp.mean((hh - mu) ** 2, axis=1, keepdims=True)
        h_s[...] = ((hh - mu) / jnp.sqrt(var + 1e-5) * hg_ref[...]
                    + hbe_ref[...])
        m_s[...] = jnp.full((S, 1), -jnp.inf, jnp.float32)
        s_s[...] = jnp.zeros((S, 1), jnp.float32)
        ll_s[...] = jnp.zeros((S, 1), jnp.float32)

    logits = jnp.dot(h_s[...], dw_ref[...],
                     preferred_element_type=jnp.float32) + db_ref[...]
    logits_ref[0] = logits
    colg = j * VT + lax.broadcasted_iota(jnp.int32, (S, VT), 1)
    lg = jnp.where(colg < V, logits, -jnp.inf)
    m_old = m_s[...]
    m_new = jnp.maximum(m_old, jnp.max(lg, axis=1, keepdims=True))
    s_s[...] = (s_s[...] * jnp.exp(m_old - m_new)
                + jnp.sum(jnp.exp(lg - m_new), axis=1, keepdims=True))
    m_s[...] = m_new
    ll_s[...] += jnp.sum(jnp.where(colg == lab_ref[...], logits, 0.0),
                         axis=1, keepdims=True)

    @pl.when(j == NV - 1)
    def _fin():
        nll = m_s[...] + jnp.log(s_s[...]) - ll_s[...]
        loss_ref[...] = jnp.sum(nll, keepdims=True) / S


def _head_pallas_args():
    row = pl.BlockSpec((1, D), lambda j: (0, 0))
    return dict(
        grid=(NV,),
        in_specs=[
            pl.BlockSpec((S, D), lambda j: (0, 0)),
            pl.BlockSpec((D, D), lambda j: (0, 0)),
            row, row, row,
            pl.BlockSpec((D, VT), lambda j: (0, j)),
            pl.BlockSpec((1, VT), lambda j: (0, j)),
            pl.BlockSpec((S, 1), lambda j: (0, 0)),
        ],
        out_specs=[
            pl.BlockSpec((1, S, VT), lambda j: (0, 0, j)),
            pl.BlockSpec((1, 1), lambda j: (0, 0)),
        ],
        out_shape=[
            jax.ShapeDtypeStruct((1, S, V), jnp.float32),
            jax.ShapeDtypeStruct((1, 1), jnp.float32),
        ],
        scratch_shapes=[
            pltpu.VMEM((S, D), jnp.float32),
            pltpu.VMEM((S, 1), jnp.float32),
            pltpu.VMEM((S, 1), jnp.float32),
            pltpu.VMEM((S, 1), jnp.float32),
        ],
    )


def _forward(gather_fn, moe, head, input_ids, attention_mask, labels,
             word_emb, pos_emb, type_emb, emb_ln_g, emb_ln_b, Wg1, bg1, Wg2,
             bg2, W1, b1, W2, b2, head_w, head_b, head_ln_g, head_ln_b,
             dec_w, dec_b):
    ids = input_ids.reshape(S)
    gathered = gather_fn(word_emb, ids)
    pos_ids = jnp.clip(jnp.arange(S) + 2, 0, pos_emb.shape[0] - 1)
    e3 = (gathered.reshape(1, S, D) + pos_emb[pos_ids][None, :, :]
          + type_emb[0])
    m = jnp.mean(e3, axis=-1, keepdims=True)
    v = jnp.var(e3, axis=-1, keepdims=True)
    x = ((e3 - m) / jnp.sqrt(v + 1e-5) * emb_ln_g + emb_ln_b).reshape(S, D)
    mask = attention_mask.reshape(S, 1).astype(jnp.float32)

    for i in range(2):
        l1 = x @ Wg1[i] + bg1[i]
        l2 = x @ Wg2[i] + bg2[i]
        scores = (l1[:, :, None] + l2[:, None, :]).reshape(S, E)
        x = moe(_bf(x), scores, W1[i], b1[i].reshape(E, 1, D), W2[i],
                b2[i].reshape(E, 1, D), mask)

    logits, loss11 = head(x, head_w, head_b.reshape(1, D),
                          head_ln_g.reshape(1, D),
                          head_ln_b.reshape(1, D), dec_w,
                          dec_b.reshape(1, V), labels.reshape(S, 1))
    return (loss11[0, 0], logits, x.reshape(1, S, D))


def kernel(input_ids, attention_mask, labels, word_emb, pos_emb, type_emb,
           emb_ln_g, emb_ln_b, Wg1, bg1, Wg2, bg2, W1, b1, W2, b2, head_w,
           head_b, head_ln_g, head_ln_b, dec_w, dec_b):
    gather_fn = _make_sc_gather()
    moe = pl.pallas_call(_moe_body, **_moe_pallas_args())
    head = pl.pallas_call(_head_body, **_head_pallas_args())
    return _forward(gather_fn, moe, head, input_ids, attention_mask,
                    labels, word_emb, pos_emb, type_emb, emb_ln_g, emb_ln_b,
                    Wg1, bg1, Wg2, bg2, W1, b1, W2, b2, head_w, head_b,
                    head_ln_g, head_ln_b, dec_w, dec_b)


# speedup vs baseline: 1.7170x; 1.5623x over previous
import functools

import jax
import jax.numpy as jnp
from jax import lax
from jax.experimental import pallas as pl
from jax.experimental.pallas import tpu as pltpu
from jax.experimental.pallas import tpu_sc as plsc

V = 30522
D = 768
G1, G2 = 2, 4
E = 8
K = 5
S = 2048
VT = 1024
NV = (V + VT - 1) // VT


def _bf(a):
    return a.astype(jnp.bfloat16)


def _make_sc_gather():
    info = plsc.get_sparse_core_info()
    nc, ns = info.num_cores, info.num_subcores
    nw = nc * ns
    b_per_w = S // nw
    mesh = plsc.VectorSubcoreMesh(core_axis_name="c", subcore_axis_name="s")

    @functools.partial(
        pl.kernel,
        out_type=jax.ShapeDtypeStruct((S, D), jnp.float32),
        mesh=mesh,
        scratch_types=[
            pltpu.VMEM((b_per_w,), jnp.int32),
            pltpu.VMEM((b_per_w, D), jnp.float32),
            pltpu.SemaphoreType.DMA,
        ],
    )
    def gather_k(table_hbm, idx_hbm, out_hbm, idx_v, rows_v, sem):
        wid = lax.axis_index("s") * nc + lax.axis_index("c")
        base = wid * b_per_w
        pltpu.sync_copy(idx_hbm.at[pl.ds(base, b_per_w)], idx_v)
        pltpu.async_copy(table_hbm.at[idx_v], rows_v, sem).wait()
        pltpu.sync_copy(rows_v, out_hbm.at[pl.ds(base, b_per_w)])

    return gather_k


def _moe_body(xb_ref, sc_ref, w1_ref, b1_ref, w2_ref, b2_ref, mask_ref,
              out_ref, gd_ref):
    e = pl.program_id(0)
    col = lax.broadcasted_iota(jnp.int32, (S, E), 1)

    @pl.when(e == 0)
    def _prologue():
        s = sc_ref[...]
        rank = jnp.zeros(s.shape, jnp.float32)
        for j in range(E):
            sj = s[:, j:j + 1]
            rank += jnp.where(sj > s, 1.0, 0.0)
            rank += jnp.where((sj == s) & (j < col), 1.0, 0.0)
        sm = jnp.where(rank < K, s, -jnp.inf)
        mx = jnp.max(sm, axis=1, keepdims=True)
        p = jnp.exp(sm - mx)
        gd_ref[...] = p / jnp.sum(p, axis=1, keepdims=True)

    h = jnp.dot(xb_ref[...], _bf(w1_ref[0]),
                preferred_element_type=jnp.float32) + b1_ref[0]
    h = jax.nn.gelu(h)
    y = jnp.dot(_bf(h), _bf(w2_ref[0]), preferred_element_type=jnp.float32) \
        + b2_ref[0]
    g = jnp.sum(jnp.where(col == e, gd_ref[...], 0.0), axis=1, keepdims=True)
    gy = _bf(g).astype(jnp.float32) * _bf(y).astype(jnp.float32)

    @pl.when(e == 0)
    def _init():
        out_ref[...] = gy

    @pl.when(e > 0)
    def _acc():
        out_ref[...] += gy

    @pl.when(e == E - 1)
    def _mask():
        out_ref[...] = out_ref[...] * mask_ref[...]


def _moe_pallas_args():
    full2d = pl.BlockSpec((S, D), lambda e: (0, 0))
    return dict(
        grid=(E,),
        in_specs=[
            full2d,
            pl.BlockSpec((S, E), lambda e: (0, 0)),
            pl.BlockSpec((1, D, D), lambda e: (e, 0, 0)),
            pl.BlockSpec((1, 1, D), lambda e: (e, 0, 0)),
            pl.BlockSpec((1, D, D), lambda e: (e, 0, 0)),
            pl.BlockSpec((1, 1, D), lambda e: (e, 0, 0)),
            pl.BlockSpec((S, 1), lambda e: (0, 0)),
        ],
        out_specs=full2d,
        out_shape=jax.ShapeDtypeStruct((S, D), jnp.float32),
        scratch_shapes=[pltpu.VMEM((S, E), jnp.float32)],
    )


def _head_body(x_ref, hw_ref, hb_ref, hg_ref, hbe_ref, dw_ref, db_ref,
               lab_ref, logits_ref, loss_ref, h_s, m_s, s_s, ll_s):
    j = pl.program_id(0)

    @pl.when(j == 0)
    def _prologue():
        hh = jnp.dot(x_ref[...], hw_ref[...],
                     preferred_element_type=jnp.float32) + hb_ref[...]
        hh = jax.nn.gelu(hh)
        mu = jnp.mean(hh, axis=1, keepdims=True)
        var = jnp.mean((hh - mu) ** 2, axis=1, keepdims=True)
        h_s[...] = _bf((hh - mu) / jnp.sqrt(var + 1e-5) * hg_ref[...]
                       + hbe_ref[...])
        m_s[...] = jnp.full((S, 1), -jnp.inf, jnp.float32)
        s_s[...] = jnp.zeros((S, 1), jnp.float32)
        ll_s[...] = jnp.zeros((S, 1), jnp.float32)

    logits = jnp.dot(h_s[...], _bf(dw_ref[...]),
                     preferred_element_type=jnp.float32) + db_ref[...]
    logits_ref[...] = logits
    coll = lax.broadcasted_iota(jnp.int32, (S, VT), 1)
    rel = lab_ref[...] - j * VT
    m_old = m_s[...]

    @pl.when(j < NV - 1)
    def _bulk():
        m_new = jnp.maximum(m_old, jnp.max(logits, axis=1, keepdims=True))
        s_s[...] = (s_s[...] * jnp.exp(m_old - m_new)
                    + jnp.sum(jnp.exp(logits - m_new), axis=1,
                              keepdims=True))
        m_s[...] = m_new

    @pl.when(j == NV - 1)
    def _last():
        lg = jnp.where(coll < V - j * VT, logits, -jnp.inf)
        m_new = jnp.maximum(m_old, jnp.max(lg, axis=1, keepdims=True))
        s_s[...] = (s_s[...] * jnp.exp(m_old - m_new)
                    + jnp.sum(jnp.exp(lg - m_new), axis=1, keepdims=True))
        m_s[...] = m_new

    ll_s[...] += jnp.sum(jnp.where(coll == rel, logits, 0.0),
                         axis=1, keepdims=True)

    @pl.when(j == NV - 1)
    def _fin():
        nll = m_s[...] + jnp.log(s_s[...]) - ll_s[...]
        loss_ref[...] = jnp.sum(nll, keepdims=True) / S


def _head_pallas_args():
    row = pl.BlockSpec((1, D), lambda j: (0, 0))
    return dict(
        grid=(NV,),
        in_specs=[
            pl.BlockSpec((S, D), lambda j: (0, 0)),
            pl.BlockSpec((D, D), lambda j: (0, 0)),
            row, row, row,
            pl.BlockSpec((D, VT), lambda j: (0, j)),
            pl.BlockSpec((1, VT), lambda j: (0, j)),
            pl.BlockSpec((S, 1), lambda j: (0, 0)),
        ],
        out_specs=[
            pl.BlockSpec((S, VT), lambda j: (0, j)),
            pl.BlockSpec((1, 1), lambda j: (0, 0)),
        ],
        out_shape=[
            jax.ShapeDtypeStruct((S, V), jnp.float32),
            jax.ShapeDtypeStruct((1, 1), jnp.float32),
        ],
        scratch_shapes=[
            pltpu.VMEM((S, D), jnp.bfloat16),
            pltpu.VMEM((S, 1), jnp.float32),
            pltpu.VMEM((S, 1), jnp.float32),
            pltpu.VMEM((S, 1), jnp.float32),
        ],
    )


def _forward(gather_fn, moe, head, input_ids, attention_mask, labels,
             word_emb, pos_emb, type_emb, emb_ln_g, emb_ln_b, Wg1, bg1, Wg2,
             bg2, W1, b1, W2, b2, head_w, head_b, head_ln_g, head_ln_b,
             dec_w, dec_b):
    ids = input_ids.reshape(S)
    gathered = gather_fn(word_emb, ids)
    pos_ids = jnp.clip(jnp.arange(S) + 2, 0, pos_emb.shape[0] - 1)
    e3 = (gathered.reshape(1, S, D) + pos_emb[pos_ids][None, :, :]
          + type_emb[0])
    m = jnp.mean(e3, axis=-1, keepdims=True)
    v = jnp.var(e3, axis=-1, keepdims=True)
    x = ((e3 - m) / jnp.sqrt(v + 1e-5) * emb_ln_g + emb_ln_b).reshape(S, D)
    mask = attention_mask.reshape(S, 1).astype(jnp.float32)

    for i in range(2):
        l1 = x @ Wg1[i] + bg1[i]
        l2 = x @ Wg2[i] + bg2[i]
        scores = (l1[:, :, None] + l2[:, None, :]).reshape(S, E)
        x = moe(_bf(x), scores, W1[i], b1[i].reshape(E, 1, D), W2[i],
                b2[i].reshape(E, 1, D), mask)

    logits, loss11 = head(x, head_w, head_b.reshape(1, D),
                          head_ln_g.reshape(1, D),
                          head_ln_b.reshape(1, D), dec_w,
                          dec_b.reshape(1, V), labels.reshape(S, 1))
    return (loss11[0, 0], logits.reshape(1, S, V), x.reshape(1, S, D))


def kernel(input_ids, attention_mask, labels, word_emb, pos_emb, type_emb,
           emb_ln_g, emb_ln_b, Wg1, bg1, Wg2, bg2, W1, b1, W2, b2, head_w,
           head_b, head_ln_g, head_ln_b, dec_w, dec_b):
    gather_fn = _make_sc_gather()
    moe = pl.pallas_call(_moe_body, **_moe_pallas_args())
    head = pl.pallas_call(_head_body, **_head_pallas_args())
    return _forward(gather_fn, moe, head, input_ids, attention_mask,
                    labels, word_emb, pos_emb, type_emb, emb_ln_g, emb_ln_b,
                    Wg1, bg1, Wg2, bg2, W1, b1, W2, b2, head_w, head_b,
                    head_ln_g, head_ln_b, dec_w, dec_b)
